# Initial kernel scaffold; baseline (speedup 1.0000x reference)
import jax
import jax.numpy as jnp
from jax.experimental import pallas as pl


def kernel(inputs, codebook):
    raise NotImplementedError("write your pallas kernel here")


# speedup vs baseline: 3.8801x; 3.8801x over previous
import functools

import jax
import jax.numpy as jnp
from jax.experimental import pallas as pl
from jax.experimental.pallas import tpu as pltpu

EMB = 32
NE = 8192
HALF = NE // 2
NTOK = 16384
BT = 256
GRID = NTOK // BT
COMMIT = 0.25
EPS = 1e-05


def _vq_kernel(xn_ref, x_ref, cn_ref, c_ref,
               qst_ref, loss_ref, perp_ref,
               counts_ref, acc_ref):
    i = pl.program_id(0)

    @pl.when(i == 0)
    def _init():
        counts_ref[...] = jnp.zeros_like(counts_ref)
        acc_ref[0] = 0.0

    x = x_ref[...]
    xn = xn_ref[...]
    cn = cn_ref[...]
    c = c_ref[...]

    mm = jax.lax.dot_general(x.astype(jnp.bfloat16), c,
                             (((1,), (1,)), ((), ())),
                             preferred_element_type=jnp.float32)
    dist = (xn + cn) - 2.0 * mm

    d1 = dist[:, :HALF]
    d2 = dist[:, HALF:]
    ids = jax.lax.broadcasted_iota(jnp.int32, (BT, HALF), 1)
    m1 = jnp.min(d1, axis=1, keepdims=True)
    i1 = jnp.min(jnp.where(d1 == m1, ids, NE), axis=1, keepdims=True)
    m1q = m1.astype(jnp.bfloat16).astype(jnp.float32)
    m2 = jnp.min(d2, axis=1, keepdims=True)
    i2 = HALF + jnp.min(jnp.where(d2 == m2, ids, NE), axis=1, keepdims=True)
    idx = jnp.where(m2 < m1q, i2, i1)

    ids_full = jax.lax.broadcasted_iota(jnp.int32, (BT, NE), 1)
    onehot = (ids_full == idx).astype(jnp.float32)

    q = jax.lax.dot_general(onehot, c, (((1,), (0,)), ((), ())),
                            preferred_element_type=jnp.float32,
                            precision=jax.lax.Precision.HIGHEST)

    qst_ref[...] = x + (q - x)
    counts_ref[...] += jnp.sum(onehot, axis=0, keepdims=True)
    acc_ref[0] += jnp.sum((q - x) ** 2)

    @pl.when(i == GRID - 1)
    def _fin():
        m = acc_ref[0] / (NTOK * EMB)
        loss_ref[...] = jnp.reshape(m + COMMIT * m, (1, 1))
        avg = counts_ref[...] * (1.0 / NTOK)
        ent = -jnp.sum(avg * jnp.log(avg + EPS))
        perp_ref[...] = jnp.reshape(jnp.exp(ent), (1, 1))


@functools.partial(jax.jit, static_argnames=())
def kernel(inputs, codebook):
    orig_shape = inputs.shape
    flat = inputs.reshape(-1, EMB)
    xn = jnp.sum(flat ** 2, axis=1, keepdims=True)
    cn = jnp.sum(codebook ** 2, axis=1).reshape(1, NE)

    qst, loss, perp = pl.pallas_call(
        _vq_kernel,
        grid=(GRID,),
        in_specs=[
            pl.BlockSpec((BT, 1), lambda i: (i, 0)),
            pl.BlockSpec((BT, EMB), lambda i: (i, 0)),
            pl.BlockSpec((1, NE), lambda i: (0, 0)),
            pl.BlockSpec((NE, EMB), lambda i: (0, 0)),
        ],
        out_specs=[
            pl.BlockSpec((BT, EMB), lambda i: (i, 0)),
            pl.BlockSpec((1, 1), lambda i: (0, 0)),
            pl.BlockSpec((1, 1), lambda i: (0, 0)),
        ],
        out_shape=[
            jax.ShapeDtypeStruct((NTOK, EMB), jnp.float32),
            jax.ShapeDtypeStruct((1, 1), jnp.float32),
            jax.ShapeDtypeStruct((1, 1), jnp.float32),
        ],
        scratch_shapes=[
            pltpu.VMEM((1, NE), jnp.float32),
            pltpu.SMEM((1,), jnp.float32),
        ],
        compiler_params=pltpu.CompilerParams(
            dimension_semantics=("arbitrary",)),
    )(xn, flat, cn, codebook)

    return (qst.reshape(orig_shape), loss.reshape(()), perp.reshape(()))


# speedup vs baseline: 4.0496x; 1.0437x over previous
import functools

import jax
import jax.numpy as jnp
from jax.experimental import pallas as pl
from jax.experimental.pallas import tpu as pltpu

EMB = 32
NE = 8192
HALF = NE // 2
NTOK = 16384
BT = 512
GRID = NTOK // BT
COMMIT = 0.25
EPS = 1e-05


def _vq_kernel(xn_ref, x_ref, cn_ref, c_ref,
               qst_ref, loss_ref, perp_ref,
               counts_ref, acc_ref):
    i = pl.program_id(0)

    @pl.when(i == 0)
    def _init():
        counts_ref[...] = jnp.zeros_like(counts_ref)
        acc_ref[0] = 0.0

    x = x_ref[...]
    xn = xn_ref[...]
    cn = cn_ref[...]
    c = c_ref[...]

    mm = jax.lax.dot_general(x.astype(jnp.bfloat16), c,
                             (((1,), (1,)), ((), ())),
                             preferred_element_type=jnp.float32)
    dist = (xn + cn) - 2.0 * mm

    d1 = dist[:, :HALF]
    d2 = dist[:, HALF:]
    ids = jax.lax.broadcasted_iota(jnp.int32, (BT, HALF), 1)
    m1 = jnp.min(d1, axis=1, keepdims=True)
    i1 = jnp.min(jnp.where(d1 == m1, ids, NE), axis=1, keepdims=True)
    m1q = m1.astype(jnp.bfloat16).astype(jnp.float32)
    m2 = jnp.min(d2, axis=1, keepdims=True)
    i2 = HALF + jnp.min(jnp.where(d2 == m2, ids, NE), axis=1, keepdims=True)
    idx = jnp.where(m2 < m1q, i2, i1)

    ids_full = jax.lax.broadcasted_iota(jnp.int32, (BT, NE), 1)
    onehot = (ids_full == idx).astype(jnp.float32)

    q = jax.lax.dot_general(onehot, c, (((1,), (0,)), ((), ())),
                            preferred_element_type=jnp.float32,
                            precision=jax.lax.Precision.HIGHEST)

    qst_ref[...] = x + (q - x)
    counts_ref[...] += jnp.sum(onehot, axis=0, keepdims=True)
    acc_ref[0] += jnp.sum((q - x) ** 2)

    @pl.when(i == GRID - 1)
    def _fin():
        m = acc_ref[0] / (NTOK * EMB)
        loss_ref[...] = jnp.reshape(m + COMMIT * m, (1, 1))
        avg = counts_ref[...] * (1.0 / NTOK)
        ent = -jnp.sum(avg * jnp.log(avg + EPS))
        perp_ref[...] = jnp.reshape(jnp.exp(ent), (1, 1))


@functools.partial(jax.jit, static_argnames=())
def kernel(inputs, codebook):
    orig_shape = inputs.shape
    flat = inputs.reshape(-1, EMB)
    xn = jnp.sum(flat ** 2, axis=1, keepdims=True)
    cn = jnp.sum(codebook ** 2, axis=1).reshape(1, NE)

    qst, loss, perp = pl.pallas_call(
        _vq_kernel,
        grid=(GRID,),
        in_specs=[
            pl.BlockSpec((BT, 1), lambda i: (i, 0)),
            pl.BlockSpec((BT, EMB), lambda i: (i, 0)),
            pl.BlockSpec((1, NE), lambda i: (0, 0)),
            pl.BlockSpec((NE, EMB), lambda i: (0, 0)),
        ],
        out_specs=[
            pl.BlockSpec((BT, EMB), lambda i: (i, 0)),
            pl.BlockSpec((1, 1), lambda i: (0, 0)),
            pl.BlockSpec((1, 1), lambda i: (0, 0)),
        ],
        out_shape=[
            jax.ShapeDtypeStruct((NTOK, EMB), jnp.float32),
            jax.ShapeDtypeStruct((1, 1), jnp.float32),
            jax.ShapeDtypeStruct((1, 1), jnp.float32),
        ],
        scratch_shapes=[
            pltpu.VMEM((1, NE), jnp.float32),
            pltpu.SMEM((1,), jnp.float32),
        ],
        compiler_params=pltpu.CompilerParams(
            dimension_semantics=("arbitrary",)),
    )(xn, flat, cn, codebook)

    return (qst.reshape(orig_shape), loss.reshape(()), perp.reshape(()))
